# baseline (speedup 1.0000x reference)
import functools

import jax
import jax.numpy as jnp
from jax import lax
from jax.experimental import pallas as pl
from jax.experimental.pallas import tpu as pltpu
from jax.experimental.pallas import tpu_sc as plsc

D_EMB = 64
BATCH = 4096
HIST = 200
NUM_WORKERS = 32
B_PER_W = BATCH // NUM_WORKERS
NBCH = 2
CHUNK = NBCH * HIST
N_CHUNKS = B_PER_W // NBCH
GGRP = 40
LANES = 16

_SPLAT_DNUMS = lax.GatherDimensionNumbers(
    offset_dims=(), collapsed_slice_dims=(0,), start_index_map=(0,))


def _splat_lane(vec, lane):
    idx = jnp.full((LANES, 1), lane, jnp.int32)
    return lax.gather(vec, idx, _SPLAT_DNUMS, slice_sizes=(1,),
                      mode=lax.GatherScatterMode.PROMISE_IN_BOUNDS)


@functools.partial(
    pl.kernel,
    mesh=plsc.VectorSubcoreMesh(core_axis_name="c", subcore_axis_name="s"),
    compiler_params=pltpu.CompilerParams(use_tc_tiling_on_sc=False,
                                         needs_layout_passes=False),
    out_type=jax.ShapeDtypeStruct((BATCH, HIST, D_EMB), jnp.float32),
    scratch_types=[
        pltpu.VMEM((CHUNK,), jnp.int32),
        pltpu.VMEM((CHUNK,), jnp.int32),
        pltpu.VMEM((CHUNK,), jnp.int32),
        pltpu.VMEM((CHUNK,), jnp.int32),
        pltpu.VMEM((CHUNK, D_EMB), jnp.float32),
        pltpu.VMEM((CHUNK, D_EMB), jnp.float32),
        pltpu.SemaphoreType.DMA,
        pltpu.SemaphoreType.DMA,
        pltpu.SemaphoreType.DMA,
        pltpu.SemaphoreType.DMA,
    ],
)
def _embed(x_ref, mask_ref, table_ref, out_ref,
           idx0, idx1, msk0, msk1, rows0, rows1,
           sem_i0, sem_i1, sem_g, sem_w):
    wid = lax.axis_index("s") * 2 + lax.axis_index("c")
    batch_w = wid * B_PER_W
    idx = (idx0, idx1)
    msk = (msk0, msk1)
    rows = (rows0, rows1)
    sem_i = (sem_i0, sem_i1)

    def stage_copies(g, b):
        base = (batch_w + g * NBCH) * HIST
        return (
            pltpu.make_async_copy(x_ref.at[pl.ds(base, CHUNK)], idx[b], sem_i[b]),
            pltpu.make_async_copy(mask_ref.at[pl.ds(base, CHUNK)], msk[b], sem_i[b]),
        )

    def gather_copies(b):
        return [
            pltpu.make_async_copy(
                table_ref.at[idx[b].at[pl.ds(i * HIST + j * GGRP, GGRP)]],
                rows[b].at[pl.ds(i * HIST + j * GGRP, GGRP)],
                sem_g,
            )
            for i in range(NBCH)
            for j in range(HIST // GGRP)
        ]

    def wb_copies(g, b):
        bb = batch_w + g * NBCH
        return [
            pltpu.make_async_copy(
                rows[b].at[pl.ds(i * HIST, HIST)], out_ref.at[bb + i], sem_w)
            for i in range(NBCH)
        ]

    def multiply(b):
        def grp_body(q, c2):
            mvec = msk[b][pl.ds(q * LANES, LANES)].astype(jnp.float32)
            for r16 in range(LANES):
                m = _splat_lane(mvec, r16)
                r = q * LANES + r16
                for s in range(D_EMB // LANES):
                    sl = rows[b][r, pl.ds(s * LANES, LANES)]
                    rows[b][r, pl.ds(s * LANES, LANES)] = sl * m
            return c2
        lax.fori_loop(0, CHUNK // LANES, grp_body, 0)

    for c in stage_copies(0, 0):
        c.start()
    for c in stage_copies(1, 1):
        c.start()
    for c in stage_copies(0, 0):
        c.wait()
    for c in gather_copies(0):
        c.start()

    def body(gi, carry):
        for b in (0, 1):
            g = 2 * gi + b
            for c in gather_copies(b):
                c.wait()
            if b == 0:
                @pl.when(gi >= 1)
                def _():
                    for c in wb_copies(g - 1, 1):
                        c.wait()
                for c in stage_copies(g + 1, 1):
                    c.wait()
                for c in gather_copies(1):
                    c.start()
            else:
                @pl.when(gi <= (N_CHUNKS - 2 - b) // 2)
                def _():
                    for c in wb_copies(g - 1, 0):
                        c.wait()
                    for c in stage_copies(g + 1, 0):
                        c.wait()
                    for c in gather_copies(0):
                        c.start()
            multiply(b)
            @pl.when(gi <= (N_CHUNKS - 3 - b) // 2)
            def _():
                for c in stage_copies(g + 2, b):
                    c.start()
            for c in wb_copies(g, b):
                c.start()
        return carry

    lax.fori_loop(0, N_CHUNKS // 2, body, 0)
    for c in wb_copies(N_CHUNKS - 2, 0):
        c.wait()
    for c in wb_copies(N_CHUNKS - 1, 1):
        c.wait()


def kernel(x, mask, table, predict):
    b, h = x.shape
    n = b * h
    xf = x.reshape(n).astype(jnp.int32)
    mf = mask.reshape(n).astype(jnp.int32)
    return _embed(xf, mf, table)


# speedup vs baseline: 1.2615x; 1.2615x over previous
import functools

import jax
import jax.numpy as jnp
from jax import lax
from jax.experimental import pallas as pl
from jax.experimental.pallas import tpu as pltpu
from jax.experimental.pallas import tpu_sc as plsc

D_EMB = 64
BATCH = 4096
HIST = 200
NUM_WORKERS = 32
B_PER_W = BATCH // NUM_WORKERS
NBCH = 2
CHUNK = NBCH * HIST
N_CHUNKS = B_PER_W // NBCH
GGRP = 40
LANES = 16

_SPLAT_DNUMS = lax.GatherDimensionNumbers(
    offset_dims=(), collapsed_slice_dims=(0,), start_index_map=(0,))


def _splat_lane(vec, lane):
    idx = jnp.full((LANES, 1), lane, jnp.int32)
    return lax.gather(vec, idx, _SPLAT_DNUMS, slice_sizes=(1,),
                      mode=lax.GatherScatterMode.PROMISE_IN_BOUNDS)


@functools.partial(
    pl.kernel,
    mesh=plsc.VectorSubcoreMesh(core_axis_name="c", subcore_axis_name="s"),
    compiler_params=pltpu.CompilerParams(use_tc_tiling_on_sc=False,
                                         needs_layout_passes=False),
    out_type=jax.ShapeDtypeStruct((BATCH, HIST, D_EMB), jnp.float32),
    scratch_types=[
        pltpu.VMEM((CHUNK,), jnp.int32),
        pltpu.VMEM((CHUNK,), jnp.int32),
        pltpu.VMEM((CHUNK,), jnp.int32),
        pltpu.VMEM((CHUNK,), jnp.int32),
        pltpu.VMEM((CHUNK, D_EMB), jnp.float32),
        pltpu.VMEM((CHUNK, D_EMB), jnp.float32),
        pltpu.SemaphoreType.DMA,
        pltpu.SemaphoreType.DMA,
        pltpu.SemaphoreType.DMA,
        pltpu.SemaphoreType.DMA,
    ],
)
def _embed(x_ref, mask_ref, table_ref, out_ref,
           idx0, idx1, msk0, msk1, rows0, rows1,
           sem_i0, sem_i1, sem_g, sem_w):
    wid = lax.axis_index("s") * 2 + lax.axis_index("c")
    batch_w = wid * B_PER_W
    idx = (idx0, idx1)
    msk = (msk0, msk1)
    rows = (rows0, rows1)
    sem_i = (sem_i0, sem_i1)

    def stage_copies(g, b):
        base = (batch_w + g * NBCH) * HIST
        return (
            pltpu.make_async_copy(x_ref.at[pl.ds(base, CHUNK)], idx[b], sem_i[b]),
            pltpu.make_async_copy(mask_ref.at[pl.ds(base, CHUNK)], msk[b], sem_i[b]),
        )

    def gather_copies(b):
        return [
            pltpu.make_async_copy(
                table_ref.at[idx[b].at[pl.ds(i * HIST + j * GGRP, GGRP)]],
                rows[b].at[pl.ds(i * HIST + j * GGRP, GGRP)],
                sem_g,
            )
            for i in range(NBCH)
            for j in range(HIST // GGRP)
        ]

    def wb_copies(g, b):
        bb = batch_w + g * NBCH
        return [
            pltpu.make_async_copy(
                rows[b].at[pl.ds(i * HIST, HIST)], out_ref.at[bb + i], sem_w)
            for i in range(NBCH)
        ]

    def multiply(b):
        @plsc.parallel_loop(0, CHUNK // LANES, unroll=2)
        def grp_body(q):
            mvec = msk[b][pl.ds(q * LANES, LANES)].astype(jnp.float32)
            for r16 in range(LANES):
                m = _splat_lane(mvec, r16)
                r = q * LANES + r16
                for s in range(D_EMB // LANES):
                    sl = rows[b][r, pl.ds(s * LANES, LANES)]
                    rows[b][r, pl.ds(s * LANES, LANES)] = sl * m

    for c in stage_copies(0, 0):
        c.start()
    for c in stage_copies(1, 1):
        c.start()
    for c in stage_copies(0, 0):
        c.wait()
    for c in gather_copies(0):
        c.start()

    def body(gi, carry):
        for b in (0, 1):
            g = 2 * gi + b
            for c in gather_copies(b):
                c.wait()
            if b == 0:
                @pl.when(gi >= 1)
                def _():
                    for c in wb_copies(g - 1, 1):
                        c.wait()
                for c in stage_copies(g + 1, 1):
                    c.wait()
                for c in gather_copies(1):
                    c.start()
            else:
                @pl.when(gi <= (N_CHUNKS - 2 - b) // 2)
                def _():
                    for c in wb_copies(g - 1, 0):
                        c.wait()
                    for c in stage_copies(g + 1, 0):
                        c.wait()
                    for c in gather_copies(0):
                        c.start()
            multiply(b)
            @pl.when(gi <= (N_CHUNKS - 3 - b) // 2)
            def _():
                for c in stage_copies(g + 2, b):
                    c.start()
            for c in wb_copies(g, b):
                c.start()
        return carry

    lax.fori_loop(0, N_CHUNKS // 2, body, 0)
    for c in wb_copies(N_CHUNKS - 2, 0):
        c.wait()
    for c in wb_copies(N_CHUNKS - 1, 1):
        c.wait()


def kernel(x, mask, table, predict):
    b, h = x.shape
    n = b * h
    xf = x.reshape(n).astype(jnp.int32)
    mf = mask.reshape(n).astype(jnp.int32)
    return _embed(xf, mf, table)


# speedup vs baseline: 1.2651x; 1.0029x over previous
import functools

import jax
import jax.numpy as jnp
from jax import lax
from jax.experimental import pallas as pl
from jax.experimental.pallas import tpu as pltpu
from jax.experimental.pallas import tpu_sc as plsc

D_EMB = 64
BATCH = 4096
HIST = 200
NUM_WORKERS = 32
B_PER_W = BATCH // NUM_WORKERS
NBCH = 2
CHUNK = NBCH * HIST
N_CHUNKS = B_PER_W // NBCH
GGRP = 40
LANES = 16

_SPLAT_DNUMS = lax.GatherDimensionNumbers(
    offset_dims=(), collapsed_slice_dims=(0,), start_index_map=(0,))


def _splat_lane(vec, lane):
    idx = jnp.full((LANES, 1), lane, jnp.int32)
    return lax.gather(vec, idx, _SPLAT_DNUMS, slice_sizes=(1,),
                      mode=lax.GatherScatterMode.PROMISE_IN_BOUNDS)


@functools.partial(
    pl.kernel,
    mesh=plsc.VectorSubcoreMesh(core_axis_name="c", subcore_axis_name="s"),
    compiler_params=pltpu.CompilerParams(use_tc_tiling_on_sc=False,
                                         needs_layout_passes=False),
    out_type=jax.ShapeDtypeStruct((BATCH, HIST, D_EMB), jnp.float32),
    scratch_types=[
        pltpu.VMEM((CHUNK,), jnp.int32),
        pltpu.VMEM((CHUNK,), jnp.int32),
        pltpu.VMEM((CHUNK,), jnp.int32),
        pltpu.VMEM((CHUNK,), jnp.int32),
        pltpu.VMEM((CHUNK, D_EMB), jnp.float32),
        pltpu.VMEM((CHUNK, D_EMB), jnp.float32),
        pltpu.SemaphoreType.DMA,
        pltpu.SemaphoreType.DMA,
        pltpu.SemaphoreType.DMA,
        pltpu.SemaphoreType.DMA,
    ],
)
def _embed(x_ref, mask_ref, table_ref, out_ref,
           idx0, idx1, msk0, msk1, rows0, rows1,
           sem_i0, sem_i1, sem_g, sem_w):
    wid = lax.axis_index("s") * 2 + lax.axis_index("c")
    batch_w = wid * B_PER_W
    idx = (idx0, idx1)
    msk = (msk0, msk1)
    rows = (rows0, rows1)
    sem_i = (sem_i0, sem_i1)

    def stage_copies(g, b):
        base = (batch_w + g * NBCH) * HIST
        return (
            pltpu.make_async_copy(x_ref.at[pl.ds(base, CHUNK)], idx[b], sem_i[b]),
            pltpu.make_async_copy(mask_ref.at[pl.ds(base, CHUNK)], msk[b], sem_i[b]),
        )

    def gather_copies(b):
        return [
            pltpu.make_async_copy(
                table_ref.at[idx[b].at[pl.ds(i * HIST + j * GGRP, GGRP)]],
                rows[b].at[pl.ds(i * HIST + j * GGRP, GGRP)],
                sem_g,
            )
            for i in range(NBCH)
            for j in range(HIST // GGRP)
        ]

    def wb_copies(g, b):
        bb = batch_w + g * NBCH
        return [
            pltpu.make_async_copy(
                rows[b].at[pl.ds(i * HIST, HIST)], out_ref.at[bb + i], sem_w)
            for i in range(NBCH)
        ]

    def multiply(b):
        @plsc.parallel_loop(0, CHUNK // LANES, unroll=2)
        def grp_body(q):
            mvec = msk[b][pl.ds(q * LANES, LANES)].astype(jnp.float32)
            for r16 in range(LANES):
                m = _splat_lane(mvec, r16)
                r = q * LANES + r16
                for s in range(D_EMB // LANES):
                    sl = rows[b][r, pl.ds(s * LANES, LANES)]
                    rows[b][r, pl.ds(s * LANES, LANES)] = sl * m

    for c in stage_copies(0, 0):
        c.start()
    for c in stage_copies(1, 1):
        c.start()
    for c in stage_copies(0, 0):
        c.wait()
    for c in gather_copies(0):
        c.start()

    def body(gi, carry):
        for b in (0, 1):
            g = 2 * gi + b
            for c in gather_copies(b):
                c.wait()
            if b == 0:
                @pl.when(gi >= 1)
                def _():
                    for c in wb_copies(g - 1, 1):
                        c.wait()
                for c in stage_copies(g + 1, 1):
                    c.wait()
                for c in gather_copies(1):
                    c.start()
            else:
                @pl.when(gi <= (N_CHUNKS - 2 - b) // 2)
                def _():
                    for c in wb_copies(g - 1, 0):
                        c.wait()
                    for c in stage_copies(g + 1, 0):
                        c.wait()
                    for c in gather_copies(0):
                        c.start()
            multiply(b)
            @pl.when(gi <= (N_CHUNKS - 3 - b) // 2)
            def _():
                for c in stage_copies(g + 2, b):
                    c.start()
            for c in wb_copies(g, b):
                c.start()
        return carry

    lax.fori_loop(0, N_CHUNKS // 2, body, 0)
    for c in wb_copies(N_CHUNKS - 2, 0):
        c.wait()
    for c in wb_copies(N_CHUNKS - 1, 1):
        c.wait()


def kernel(x, mask, table, predict):
    b, h = x.shape
    n = b * h
    xf = x.reshape(n).astype(jnp.int32)
    mf = mask.reshape(n).astype(jnp.int32)
    tf = lax.optimization_barrier(table.reshape(-1))
    out = _embed(xf, mf, tf.reshape(table.shape))
    return lax.optimization_barrier(out.reshape(-1)).reshape(out.shape)


# speedup vs baseline: 1.2879x; 1.0180x over previous
import functools

import jax
import jax.numpy as jnp
from jax import lax
from jax.experimental import pallas as pl
from jax.experimental.pallas import tpu as pltpu
from jax.experimental.pallas import tpu_sc as plsc

D_EMB = 64
BATCH = 4096
HIST = 200
NUM_WORKERS = 32
B_PER_W = BATCH // NUM_WORKERS
N_CHUNKS = B_PER_W
GGRP = 40
LANES = 16
NGRP_FULL = HIST // LANES
TAIL = HIST - LANES
IBUF = 256

_SPLAT_DNUMS = lax.GatherDimensionNumbers(
    offset_dims=(), collapsed_slice_dims=(0,), start_index_map=(0,))


def _splat_lane(vec, lane):
    idx = jnp.full((LANES, 1), lane, jnp.int32)
    return lax.gather(vec, idx, _SPLAT_DNUMS, slice_sizes=(1,),
                      mode=lax.GatherScatterMode.PROMISE_IN_BOUNDS)


@functools.partial(
    pl.kernel,
    mesh=plsc.VectorSubcoreMesh(core_axis_name="c", subcore_axis_name="s"),
    compiler_params=pltpu.CompilerParams(use_tc_tiling_on_sc=True),
    out_type=jax.ShapeDtypeStruct((BATCH, HIST, D_EMB), jnp.float32),
    scratch_types=[
        pltpu.VMEM((IBUF,), jnp.int32),
        pltpu.VMEM((IBUF,), jnp.int32),
        pltpu.VMEM((IBUF,), jnp.int32),
        pltpu.VMEM((IBUF,), jnp.int32),
        pltpu.VMEM((IBUF,), jnp.int32),
        pltpu.VMEM((IBUF,), jnp.int32),
        pltpu.VMEM((HIST, 2 * D_EMB), jnp.float32),
        pltpu.VMEM((HIST, 2 * D_EMB), jnp.float32),
        pltpu.VMEM((HIST, D_EMB), jnp.float32),
        pltpu.VMEM((HIST, D_EMB), jnp.float32),
        pltpu.SemaphoreType.DMA,
        pltpu.SemaphoreType.DMA,
        pltpu.SemaphoreType.DMA,
        pltpu.SemaphoreType.DMA,
    ],
)
def _embed(x_ref, mask_ref, table_ref, out_ref,
           idx0, idx1, msk0, msk1, pdx0, pdx1, rows0, rows1, res0, res1,
           sem_i0, sem_i1, sem_g, sem_w):
    wid = lax.axis_index("s") * 2 + lax.axis_index("c")
    batch_w = wid * B_PER_W
    idx = (idx0, idx1)
    msk = (msk0, msk1)
    pdx = (pdx0, pdx1)
    rows = (rows0, rows1)
    res = (res0, res1)
    sem_i = (sem_i0, sem_i1)

    def stage_copies(g, b):
        base = (batch_w + g) * HIST
        return (
            pltpu.make_async_copy(
                x_ref.at[pl.ds(base, HIST)], idx[b].at[pl.ds(0, HIST)], sem_i[b]),
            pltpu.make_async_copy(
                mask_ref.at[pl.ds(base, HIST)], msk[b].at[pl.ds(0, HIST)], sem_i[b]),
        )

    def comp_pidx(b):
        @plsc.parallel_loop(0, NGRP_FULL + 1, unroll=2)
        def _(q):
            pdx[b][pl.ds(q * LANES, LANES)] = (
                idx[b][pl.ds(q * LANES, LANES)] >> 1)

    def gather_copies(b):
        return [
            pltpu.make_async_copy(
                table_ref.at[pdx[b].at[pl.ds(j * GGRP, GGRP)]],
                rows[b].at[pl.ds(j * GGRP, GGRP)],
                sem_g,
            )
            for j in range(HIST // GGRP)
        ]

    def wb_copy(g, b):
        return pltpu.make_async_copy(res[b], out_ref.at[batch_w + g], sem_w)

    def do_group(b, start):
        code = (idx[b][pl.ds(start, LANES)] & 1) + 2 * msk[b][pl.ds(start, LANES)]
        for r16 in range(LANES):
            cs = _splat_lane(code, r16)
            of = (cs & 1).astype(jnp.float32)
            mf = (cs >> 1).astype(jnp.float32)
            w_hi = mf * of
            w_lo = mf - w_hi
            r = start + r16
            for s in range(D_EMB // LANES):
                lo = rows[b][r, pl.ds(s * LANES, LANES)]
                hi = rows[b][r, pl.ds(D_EMB + s * LANES, LANES)]
                res[b][r, pl.ds(s * LANES, LANES)] = lo * w_lo + hi * w_hi

    def process(b):
        @plsc.parallel_loop(0, NGRP_FULL, unroll=2)
        def _(q):
            do_group(b, q * LANES)
        do_group(b, TAIL)

    for c in stage_copies(0, 0):
        c.start()
    for c in stage_copies(1, 1):
        c.start()
    for c in stage_copies(0, 0):
        c.wait()
    comp_pidx(0)
    for c in gather_copies(0):
        c.start()

    def body(gi, carry):
        for b in (0, 1):
            g = 2 * gi + b
            for c in gather_copies(b):
                c.wait()
            if b == 0:
                @pl.when(gi >= 1)
                def _():
                    wb_copy(g - 1, 1).wait()
                for c in stage_copies(g + 1, 1):
                    c.wait()
                comp_pidx(1)
                for c in gather_copies(1):
                    c.start()
            else:
                @pl.when(gi <= (N_CHUNKS - 2 - b) // 2)
                def _():
                    wb_copy(g - 1, 0).wait()
                    for c in stage_copies(g + 1, 0):
                        c.wait()
                    comp_pidx(0)
                    for c in gather_copies(0):
                        c.start()
            process(b)
            @pl.when(gi <= (N_CHUNKS - 3 - b) // 2)
            def _():
                for c in stage_copies(g + 2, b):
                    c.start()
            wb_copy(g, b).start()
        return carry

    lax.fori_loop(0, N_CHUNKS // 2, body, 0)
    wb_copy(N_CHUNKS - 2, 0).wait()
    wb_copy(N_CHUNKS - 1, 1).wait()


def kernel(x, mask, table, predict):
    b, h = x.shape
    n = b * h
    xf = x.reshape(n).astype(jnp.int32)
    mf = mask.reshape(n).astype(jnp.int32)
    table2 = table.reshape(table.shape[0] // 2, 2 * table.shape[1])
    return _embed(xf, mf, table2)


# speedup vs baseline: 1.3780x; 1.0700x over previous
import functools

import jax
import jax.numpy as jnp
from jax import lax
from jax.experimental import pallas as pl
from jax.experimental.pallas import tpu as pltpu
from jax.experimental.pallas import tpu_sc as plsc

D_EMB = 64
BATCH = 4096
HIST = 200
NUM_WORKERS = 32
B_PER_W = BATCH // NUM_WORKERS
N_CHUNKS = B_PER_W
GGRP = 40
LANES = 16
NGRP_FULL = HIST // LANES
TAIL = HIST - LANES
IBUF = 256

_SPLAT_DNUMS = lax.GatherDimensionNumbers(
    offset_dims=(), collapsed_slice_dims=(0,), start_index_map=(0,))


def _splat_lane(vec, lane):
    idx = jnp.full((LANES, 1), lane, jnp.int32)
    return lax.gather(vec, idx, _SPLAT_DNUMS, slice_sizes=(1,),
                      mode=lax.GatherScatterMode.PROMISE_IN_BOUNDS)


@functools.partial(
    pl.kernel,
    mesh=plsc.VectorSubcoreMesh(core_axis_name="c", subcore_axis_name="s"),
    compiler_params=pltpu.CompilerParams(use_tc_tiling_on_sc=True),
    out_type=jax.ShapeDtypeStruct((BATCH, HIST, D_EMB), jnp.float32),
    scratch_types=[
        pltpu.VMEM((IBUF,), jnp.int32),
        pltpu.VMEM((IBUF,), jnp.int32),
        pltpu.VMEM((IBUF,), jnp.int32),
        pltpu.VMEM((IBUF,), jnp.int32),
        pltpu.VMEM((HIST, 2 * D_EMB), jnp.float32),
        pltpu.VMEM((HIST, 2 * D_EMB), jnp.float32),
        pltpu.VMEM((HIST, D_EMB), jnp.float32),
        pltpu.VMEM((HIST, D_EMB), jnp.float32),
        pltpu.SemaphoreType.DMA,
        pltpu.SemaphoreType.DMA,
        pltpu.SemaphoreType.DMA,
        pltpu.SemaphoreType.DMA,
    ],
)
def _embed(x_ref, mask_ref, table_ref, out_ref,
           idx0, idx1, msk0, msk1, rows0, rows1, res0, res1,
           sem_i0, sem_i1, sem_g, sem_w):
    wid = lax.axis_index("s") * 2 + lax.axis_index("c")
    batch_w = wid * B_PER_W
    idx = (idx0, idx1)
    msk = (msk0, msk1)
    rows = (rows0, rows1)
    res = (res0, res1)
    sem_i = (sem_i0, sem_i1)

    def stage_copies(g, b):
        base = (batch_w + g) * HIST
        return (
            pltpu.make_async_copy(
                x_ref.at[pl.ds(base, HIST)], idx[b].at[pl.ds(0, HIST)], sem_i[b]),
            pltpu.make_async_copy(
                mask_ref.at[pl.ds(base, HIST)], msk[b].at[pl.ds(0, HIST)], sem_i[b]),
        )

    def gather_copies(b):
        return [
            pltpu.make_async_copy(
                table_ref.at[idx[b].at[pl.ds(j * GGRP, GGRP)]],
                rows[b].at[pl.ds(j * GGRP, GGRP)],
                sem_g,
            )
            for j in range(HIST // GGRP)
        ]

    def wb_copy(g, b):
        return pltpu.make_async_copy(res[b], out_ref.at[batch_w + g], sem_w)

    def do_group(b, start):
        code = msk[b][pl.ds(start, LANES)]
        for r16 in range(LANES):
            mf = _splat_lane(code, r16).astype(jnp.float32)
            r = start + r16
            for s in range(D_EMB // LANES):
                sl = rows[b][r, pl.ds(s * LANES, LANES)]
                res[b][r, pl.ds(s * LANES, LANES)] = sl * mf

    def process(b):
        @plsc.parallel_loop(0, NGRP_FULL, unroll=2)
        def _(q):
            do_group(b, q * LANES)
        do_group(b, TAIL)

    for c in stage_copies(0, 0):
        c.start()
    for c in stage_copies(1, 1):
        c.start()
    for c in stage_copies(0, 0):
        c.wait()
    for c in gather_copies(0):
        c.start()

    def body(gi, carry):
        for b in (0, 1):
            g = 2 * gi + b
            for c in gather_copies(b):
                c.wait()
            if b == 0:
                @pl.when(gi >= 1)
                def _():
                    wb_copy(g - 1, 1).wait()
                for c in stage_copies(g + 1, 1):
                    c.wait()
                for c in gather_copies(1):
                    c.start()
            else:
                @pl.when(gi <= (N_CHUNKS - 2 - b) // 2)
                def _():
                    wb_copy(g - 1, 0).wait()
                    for c in stage_copies(g + 1, 0):
                        c.wait()
                    for c in gather_copies(0):
                        c.start()
            process(b)
            @pl.when(gi <= (N_CHUNKS - 3 - b) // 2)
            def _():
                for c in stage_copies(g + 2, b):
                    c.start()
            wb_copy(g, b).start()
        return carry

    lax.fori_loop(0, N_CHUNKS // 2, body, 0)
    wb_copy(N_CHUNKS - 2, 0).wait()
    wb_copy(N_CHUNKS - 1, 1).wait()


def kernel(x, mask, table, predict):
    b, h = x.shape
    n = b * h
    xf = x.reshape(n).astype(jnp.int32)
    mf = mask.reshape(n).astype(jnp.int32)
    table2 = jnp.pad(table, ((0, 0), (0, D_EMB)))
    return _embed(xf, mf, table2)
